# baseline (speedup 1.0000x reference)
import jax
import jax.numpy as jnp
from jax.experimental import pallas as pl
from jax.experimental.pallas import tpu as pltpu


def _copy3_kernel(u_ref, i_ref, t_ref, ou_ref, oi_ref, ot_ref):
    def scoped(s_u, s_i, s_t):
        cu = pltpu.make_async_copy(u_ref, ou_ref, s_u)
        ci = pltpu.make_async_copy(i_ref, oi_ref, s_i)
        ct = pltpu.make_async_copy(t_ref, ot_ref, s_t)
        cu.start()
        ci.start()
        ct.start()
        cu.wait()
        ci.wait()
        ct.wait()

    pl.run_scoped(
        scoped,
        pltpu.SemaphoreType.DMA,
        pltpu.SemaphoreType.DMA,
        pltpu.SemaphoreType.DMA,
    )


def kernel(embed_user, embed_item, embed_tag):
    any_spec = pl.BlockSpec(memory_space=pl.ANY)
    out = pl.pallas_call(
        _copy3_kernel,
        in_specs=[any_spec, any_spec, any_spec],
        out_specs=[any_spec, any_spec, any_spec],
        out_shape=[
            jax.ShapeDtypeStruct(embed_user.shape, embed_user.dtype),
            jax.ShapeDtypeStruct(embed_item.shape, embed_item.dtype),
            jax.ShapeDtypeStruct(embed_tag.shape, embed_tag.dtype),
        ],
    )(embed_user, embed_item, embed_tag)
    return tuple(out)


# speedup vs baseline: 45.1402x; 45.1402x over previous
import jax
import jax.numpy as jnp
from jax.experimental import pallas as pl
from jax.experimental.pallas import tpu as pltpu


_STEPS = 50


def _copy3_kernel(u_ref, i_ref, t_ref, ou_ref, oi_ref, ot_ref):
    ou_ref[...] = u_ref[...]
    oi_ref[...] = i_ref[...]
    ot_ref[...] = t_ref[...]


def kernel(embed_user, embed_item, embed_tag):
    nu, d = embed_user.shape
    ni, _ = embed_item.shape
    nt, _ = embed_tag.shape
    bu, bi, bt = nu // _STEPS, ni // _STEPS, nt // _STEPS

    def spec(block_rows):
        return pl.BlockSpec((block_rows, d), lambda s: (s, 0))

    out = pl.pallas_call(
        _copy3_kernel,
        grid=(_STEPS,),
        in_specs=[spec(bu), spec(bi), spec(bt)],
        out_specs=[spec(bu), spec(bi), spec(bt)],
        out_shape=[
            jax.ShapeDtypeStruct(embed_user.shape, embed_user.dtype),
            jax.ShapeDtypeStruct(embed_item.shape, embed_item.dtype),
            jax.ShapeDtypeStruct(embed_tag.shape, embed_tag.dtype),
        ],
    )(embed_user, embed_item, embed_tag)
    return tuple(out)


# speedup vs baseline: 45.2431x; 1.0023x over previous
import jax
import jax.numpy as jnp
from jax.experimental import pallas as pl
from jax.experimental.pallas import tpu as pltpu


_STEPS = 50


def _copy3_kernel(u_ref, i_ref, t_ref, ou_ref, oi_ref, ot_ref):
    ou_ref[...] = u_ref[...]
    oi_ref[...] = i_ref[...]
    ot_ref[...] = t_ref[...]


def kernel(embed_user, embed_item, embed_tag):
    nu, d = embed_user.shape
    ni, _ = embed_item.shape
    nt, _ = embed_tag.shape
    bu, bi, bt = nu // _STEPS, ni // _STEPS, nt // _STEPS

    def spec(block_rows):
        return pl.BlockSpec((block_rows, d), lambda s: (s, 0))

    out = pl.pallas_call(
        _copy3_kernel,
        grid=(_STEPS,),
        compiler_params=pltpu.CompilerParams(dimension_semantics=("parallel",)),
        in_specs=[spec(bu), spec(bi), spec(bt)],
        out_specs=[spec(bu), spec(bi), spec(bt)],
        out_shape=[
            jax.ShapeDtypeStruct(embed_user.shape, embed_user.dtype),
            jax.ShapeDtypeStruct(embed_item.shape, embed_item.dtype),
            jax.ShapeDtypeStruct(embed_tag.shape, embed_tag.dtype),
        ],
    )(embed_user, embed_item, embed_tag)
    return tuple(out)
